# baseline (speedup 1.0000x reference)
import jax
import jax.numpy as jnp
from jax import lax
from jax.experimental import pallas as pl

B, D, C = 16384, 1024, 51
NSTREAM = 4
BLK = 512


def _head(x_ref, w_ref, b_ref, bins_ref, probs_ref, val_ref, row0):
    lt = lax.dot_general(
        w_ref[...], x_ref[...],
        (((1,), (1,)), ((), ())),
        preferred_element_type=jnp.float32,
    )
    lt = lt + b_ref[...]
    m = jnp.max(lt, axis=0, keepdims=True)
    e = jnp.exp(lt - m)
    s = jnp.sum(e, axis=0, keepdims=True)
    rinv = 1.0 / s
    num = jnp.sum(e * bins_ref[...], axis=0, keepdims=True)
    pt = e * rinv
    blk = x_ref.shape[0]
    probs_ref[row0:row0 + blk, :] = pt.T
    val_ref[0, 0, row0:row0 + blk] = (num * rinv)[0, :]


def _head_kernel(*refs):
    x_refs = refs[:NSTREAM]
    w_ref, b_ref, bins_ref, probs_ref, val_ref = refs[NSTREAM:]
    for j in range(NSTREAM):
        _head(x_refs[j], w_ref, b_ref, bins_ref, probs_ref, val_ref, j * BLK)


def _mk_in_spec(j):
    return pl.BlockSpec((BLK, D), lambda i, j=j: (NSTREAM * i + j, 0))


@jax.jit
def kernel(x, W, b, bins):
    rows = NSTREAM * BLK
    ng = B // rows
    b2 = b.reshape(C, 1)
    bins2 = bins.reshape(C, 1)
    probs, val = pl.pallas_call(
        _head_kernel,
        grid=(ng,),
        in_specs=[_mk_in_spec(j) for j in range(NSTREAM)] + [
            pl.BlockSpec((C, D), lambda i: (0, 0)),
            pl.BlockSpec((C, 1), lambda i: (0, 0)),
            pl.BlockSpec((C, 1), lambda i: (0, 0)),
        ],
        out_specs=[
            pl.BlockSpec((rows, C), lambda i: (i, 0)),
            pl.BlockSpec((1, 1, rows), lambda i: (i, 0, 0)),
        ],
        out_shape=[
            jax.ShapeDtypeStruct((B, C), jnp.float32),
            jax.ShapeDtypeStruct((ng, 1, rows), jnp.float32),
        ],
    )(*([x] * NSTREAM), W, b2, bins2)
    return probs, val.reshape(B)


# speedup vs baseline: 1.0033x; 1.0033x over previous
import jax
import jax.numpy as jnp
from jax import lax
from jax.experimental import pallas as pl
from jax.experimental.pallas import tpu as pltpu

B, D, C = 16384, 1024, 51
BLK = 1024
NB = B // BLK
NBUF = 4


def _in_copy(x_hbm, xbuf, insem, i):
    return pltpu.make_async_copy(
        x_hbm.at[pl.ds(i * BLK, BLK), :], xbuf.at[i % NBUF], insem.at[i % NBUF])


def _out_copy(pbuf, probs_hbm, outsem, i):
    return pltpu.make_async_copy(
        pbuf.at[i % NBUF], probs_hbm.at[pl.ds(i * BLK, BLK), :],
        outsem.at[i % NBUF])


def _head_kernel(x_hbm, w_ref, b_ref, bins_ref, probs_hbm, val_ref,
                 xbuf, pbuf, insem, outsem):
    for j in range(NBUF):
        _in_copy(x_hbm, xbuf, insem, j).start()
    for i in range(NB):
        j = i % NBUF
        _in_copy(x_hbm, xbuf, insem, i).wait()
        if i >= NBUF:
            _out_copy(pbuf, probs_hbm, outsem, i - NBUF).wait()
        lt = lax.dot_general(
            w_ref[...], xbuf[j],
            (((1,), (1,)), ((), ())),
            preferred_element_type=jnp.float32,
        )
        lt = lt + b_ref[...]
        m = jnp.max(lt, axis=0, keepdims=True)
        e = jnp.exp(lt - m)
        s = jnp.sum(e, axis=0, keepdims=True)
        rinv = 1.0 / s
        num = jnp.sum(e * bins_ref[...], axis=0, keepdims=True)
        pbuf[j] = (e * rinv).T
        val_ref[0, pl.ds(i * BLK, BLK)] = (num * rinv)[0, :]
        _out_copy(pbuf, probs_hbm, outsem, i).start()
        if i + NBUF < NB:
            _in_copy(x_hbm, xbuf, insem, i + NBUF).start()
    for i in range(NB - NBUF, NB):
        _out_copy(pbuf, probs_hbm, outsem, i).wait()


@jax.jit
def kernel(x, W, b, bins):
    b2 = b.reshape(C, 1)
    bins2 = bins.reshape(C, 1)
    probs, val = pl.pallas_call(
        _head_kernel,
        in_specs=[
            pl.BlockSpec(memory_space=pltpu.HBM),
            pl.BlockSpec(memory_space=pltpu.VMEM),
            pl.BlockSpec(memory_space=pltpu.VMEM),
            pl.BlockSpec(memory_space=pltpu.VMEM),
        ],
        out_specs=[
            pl.BlockSpec(memory_space=pltpu.HBM),
            pl.BlockSpec(memory_space=pltpu.VMEM),
        ],
        out_shape=[
            jax.ShapeDtypeStruct((B, C), jnp.float32),
            jax.ShapeDtypeStruct((1, B), jnp.float32),
        ],
        scratch_shapes=[
            pltpu.VMEM((NBUF, BLK, D), jnp.float32),
            pltpu.VMEM((NBUF, BLK, C), jnp.float32),
            pltpu.SemaphoreType.DMA((NBUF,)),
            pltpu.SemaphoreType.DMA((NBUF,)),
        ],
    )(x, W, b2, bins2)
    return probs, val.reshape(B)
